# baseline (speedup 1.0000x reference)
import functools

import jax
import jax.numpy as jnp
from jax.experimental import pallas as pl
from jax.experimental.pallas import tpu as pltpu


def _vq_block_kernel(x_ref, cbt_ref, cb_ref, tok_ref, q_ref):
    x = x_ref[...]
    cbt = cbt_ref[...]
    p = jnp.dot(x, cbt, preferred_element_type=jnp.float32)
    x_sq = jnp.sum(x * x, axis=1, keepdims=True)
    cb = cb_ref[...]
    k_sq = jnp.sum(cb * cb, axis=1)
    d = (x_sq - 2.0 * p) + k_sq[None, :]
    m = jnp.min(d, axis=1, keepdims=True)
    kk = d.shape[1]
    iota = jax.lax.broadcasted_iota(jnp.int32, d.shape, 1)
    tok = jnp.min(jnp.where(d == m, iota, kk), axis=1)
    tok_ref[...] = tok
    oh = (iota == tok[:, None]).astype(jnp.float32)
    q_ref[...] = jnp.dot(oh, cb, preferred_element_type=jnp.float32,
                         precision=jax.lax.Precision.HIGHEST)


def kernel(z, codebook):
    b, d, t = z.shape
    k = codebook.shape[0]
    n = b * t
    x = jnp.transpose(z, (0, 2, 1)).reshape(n, d)
    cbt = codebook.T

    bn = 512
    grid = (n // bn,)
    tok, q = pl.pallas_call(
        _vq_block_kernel,
        grid=grid,
        in_specs=[
            pl.BlockSpec((bn, d), lambda i: (i, 0)),
            pl.BlockSpec((d, k), lambda i: (0, 0)),
            pl.BlockSpec((k, d), lambda i: (0, 0)),
        ],
        out_specs=[
            pl.BlockSpec((bn,), lambda i: (i,)),
            pl.BlockSpec((bn, d), lambda i: (i, 0)),
        ],
        out_shape=[
            jax.ShapeDtypeStruct((n,), jnp.int32),
            jax.ShapeDtypeStruct((n, d), jnp.float32),
        ],
    )(x, cbt, codebook)
    return tok.reshape(b, t), q.reshape(b, t, d)


# speedup vs baseline: 2.0687x; 2.0687x over previous
import functools

import jax
import jax.numpy as jnp
from jax import lax
from jax.experimental import pallas as pl
from jax.experimental.pallas import tpu as pltpu
from jax.experimental.pallas import tpu_sc as plsc


def _tok_block_kernel(x_ref, cbt_ref, cb_ref, tok_ref):
    x = x_ref[...]
    cbt = cbt_ref[...]
    p = jnp.dot(x, cbt, preferred_element_type=jnp.float32)
    x_sq = jnp.sum(x * x, axis=1, keepdims=True)
    cb = cb_ref[...]
    k_sq = jnp.sum(cb * cb, axis=1)
    d = (x_sq - 2.0 * p) + k_sq[None, :]
    m = jnp.min(d, axis=1, keepdims=True)
    kk = d.shape[1]
    iota = jax.lax.broadcasted_iota(jnp.int32, d.shape, 1)
    tok_ref[...] = jnp.min(jnp.where(d == m, iota, kk), axis=1)


def _make_sc_gather(n, d, k):
    info = plsc.get_sparse_core_info()
    nc, ns, nl = info.num_cores, info.num_subcores, info.num_lanes
    nw = nc * ns
    assert d % nl == 0 and n % (8 * nw) == 0
    b_per_w = n // nw
    mesh = plsc.VectorSubcoreMesh(core_axis_name="c", subcore_axis_name="s")

    @functools.partial(
        pl.kernel,
        mesh=mesh,
        out_type=jax.ShapeDtypeStruct((n, 128), jnp.float32),
        scratch_types=[
            pltpu.VMEM((b_per_w,), jnp.int32),
            pltpu.VMEM((b_per_w // 4, 128), jnp.float32),
            pltpu.VMEM((b_per_w // 4, 128), jnp.float32),
            pltpu.SemaphoreType.DMA,
            pltpu.SemaphoreType.DMA,
        ],
    )
    def gather(tok_hbm, cb_hbm, out_hbm, idx_v, rows0, rows1, sem0, sem1):
        wid = lax.axis_index("s") * nc + lax.axis_index("c")
        base = wid * b_per_w
        chunk = b_per_w // 4
        pltpu.sync_copy(tok_hbm.at[pl.ds(base, b_per_w)], idx_v)
        rows = (rows0, rows1)
        sems = (sem0, sem1)
        cps = []
        for c in range(4):
            if c >= 2:
                cps[c - 2].wait()
                pltpu.sync_copy(rows[c % 2],
                                out_hbm.at[pl.ds(base + (c - 2) * chunk, chunk)])
            cps.append(pltpu.async_copy(
                cb_hbm.at[idx_v.at[pl.ds(c * chunk, chunk)]],
                rows[c % 2], sems[c % 2]))
        for c in range(2, 4):
            cps[c].wait()
            pltpu.sync_copy(rows[c % 2],
                            out_hbm.at[pl.ds(base + c * chunk, chunk)])

    return gather


def kernel(z, codebook):
    b, d, t = z.shape
    k = codebook.shape[0]
    n = b * t
    x = jnp.transpose(z, (0, 2, 1)).reshape(n, d)
    cbt = codebook.T

    bn = 512
    tok = pl.pallas_call(
        _tok_block_kernel,
        grid=(n // bn,),
        in_specs=[
            pl.BlockSpec((bn, d), lambda i: (i, 0)),
            pl.BlockSpec((d, k), lambda i: (0, 0)),
            pl.BlockSpec((k, d), lambda i: (0, 0)),
        ],
        out_specs=pl.BlockSpec((bn,), lambda i: (i,)),
        out_shape=jax.ShapeDtypeStruct((n,), jnp.int32),
    )(x, cbt, codebook)

    cb_pad = jnp.pad(codebook, ((0, 0), (0, 128 - d)))
    q = _make_sc_gather(n, d, k)(tok, cb_pad)[:, :d]
    return tok.reshape(b, t), q.reshape(b, t, d)


# speedup vs baseline: 2.2752x; 1.0998x over previous
import functools

import jax
import jax.numpy as jnp
from jax import lax
from jax.experimental import pallas as pl
from jax.experimental.pallas import tpu as pltpu
from jax.experimental.pallas import tpu_sc as plsc


def _tok_block_kernel(x_ref, cbt_ref, cb_ref, tok_ref, ksq_ref):
    @pl.when(pl.program_id(0) == 0)
    def _():
        cb = cb_ref[...]
        ksq_ref[...] = jnp.sum(cb * cb, axis=1)[None, :]

    x = x_ref[...]
    cbt = cbt_ref[...]
    p = jnp.dot(x, cbt, preferred_element_type=jnp.float32)
    x_sq = jnp.sum(x * x, axis=1, keepdims=True)
    d = (x_sq - 2.0 * p) + ksq_ref[...]
    m = jnp.min(d, axis=1, keepdims=True)
    kk = d.shape[1]
    iota = jax.lax.broadcasted_iota(jnp.int32, d.shape, 1)
    tok_ref[...] = jnp.min(jnp.where(d == m, iota, kk), axis=1)


def _make_sc_gather(n, d, k):
    info = plsc.get_sparse_core_info()
    nc, ns, nl = info.num_cores, info.num_subcores, info.num_lanes
    nw = nc * ns
    assert d % nl == 0 and n % (8 * nw) == 0
    b_per_w = n // nw
    mesh = plsc.VectorSubcoreMesh(core_axis_name="c", subcore_axis_name="s")

    @functools.partial(
        pl.kernel,
        mesh=mesh,
        out_type=jax.ShapeDtypeStruct((n, 128), jnp.float32),
        scratch_types=[
            pltpu.VMEM((b_per_w,), jnp.int32),
            pltpu.VMEM((b_per_w // 4, 128), jnp.float32),
            pltpu.VMEM((b_per_w // 4, 128), jnp.float32),
            pltpu.SemaphoreType.DMA,
            pltpu.SemaphoreType.DMA,
        ],
    )
    def gather(tok_hbm, cb_hbm, out_hbm, idx_v, rows0, rows1, sem0, sem1):
        wid = lax.axis_index("s") * nc + lax.axis_index("c")
        base = wid * b_per_w
        chunk = b_per_w // 4
        pltpu.sync_copy(tok_hbm.at[pl.ds(base, b_per_w)], idx_v)
        rows = (rows0, rows1)
        sems = (sem0, sem1)
        cps = []
        for c in range(4):
            if c >= 2:
                cps[c - 2].wait()
                pltpu.sync_copy(rows[c % 2],
                                out_hbm.at[pl.ds(base + (c - 2) * chunk, chunk)])
            cps.append(pltpu.async_copy(
                cb_hbm.at[idx_v.at[pl.ds(c * chunk, chunk)]],
                rows[c % 2], sems[c % 2]))
        for c in range(2, 4):
            cps[c].wait()
            pltpu.sync_copy(rows[c % 2],
                            out_hbm.at[pl.ds(base + c * chunk, chunk)])

    return gather


def kernel(z, codebook):
    b, d, t = z.shape
    k = codebook.shape[0]
    n = b * t
    x = jnp.transpose(z, (0, 2, 1)).reshape(n, d)
    cbt = codebook.T

    bn = 1024
    tok = pl.pallas_call(
        _tok_block_kernel,
        grid=(n // bn,),
        in_specs=[
            pl.BlockSpec((bn, d), lambda i: (i, 0)),
            pl.BlockSpec((d, k), lambda i: (0, 0)),
            pl.BlockSpec((k, d), lambda i: (0, 0)),
        ],
        out_specs=pl.BlockSpec((bn,), lambda i: (i,)),
        out_shape=jax.ShapeDtypeStruct((n,), jnp.int32),
        scratch_shapes=[pltpu.VMEM((1, k), jnp.float32)],
    )(x, cbt, codebook)

    cb_pad = jnp.pad(codebook, ((0, 0), (0, 128 - d)))
    q = _make_sc_gather(n, d, k)(tok, cb_pad)[:, :d]
    return tok.reshape(b, t), q.reshape(b, t, d)


# speedup vs baseline: 3.0212x; 1.3279x over previous
import functools

import jax
import jax.numpy as jnp
from jax import lax
from jax.experimental import pallas as pl
from jax.experimental.pallas import tpu as pltpu
from jax.experimental.pallas import tpu_sc as plsc


def _tok_block_kernel(z_ref, cb_ref, tok_ref, ksq_ref):
    @pl.when((pl.program_id(0) == 0) & (pl.program_id(1) == 0))
    def _():
        cb0 = cb_ref[...]
        ksq_ref[...] = jnp.sum(cb0 * cb0, axis=1, keepdims=True)

    zb = z_ref[0]
    p = jnp.dot(cb_ref[...], zb, preferred_element_type=jnp.float32)
    s = zb * zb
    w = s.shape[0] // 2
    while w >= 1:
        s = (jax.lax.slice_in_dim(s, 0, w, axis=0)
             + jax.lax.slice_in_dim(s, w, 2 * w, axis=0))
        w //= 2
    d = (s - 2.0 * p) + ksq_ref[...]
    m = jnp.min(d, axis=0, keepdims=True)
    kk = d.shape[0]
    iota0 = jax.lax.broadcasted_iota(
        jnp.int32, d.shape, 0).astype(jnp.float32)
    tok_ref[0] = jnp.min(jnp.where(d == m, iota0, float(kk)), axis=0,
                         keepdims=True).astype(jnp.int32)


def _make_sc_gather(n, d, k):
    info = plsc.get_sparse_core_info()
    nc, ns, nl = info.num_cores, info.num_subcores, info.num_lanes
    nw = nc * ns
    assert d % nl == 0 and n % (8 * nw) == 0
    b_per_w = n // nw
    mesh = plsc.VectorSubcoreMesh(core_axis_name="c", subcore_axis_name="s")

    @functools.partial(
        pl.kernel,
        mesh=mesh,
        out_type=jax.ShapeDtypeStruct((n, 128), jnp.float32),
        scratch_types=[
            pltpu.VMEM((b_per_w,), jnp.int32),
            pltpu.VMEM((b_per_w // 4, 128), jnp.float32),
            pltpu.VMEM((b_per_w // 4, 128), jnp.float32),
            pltpu.SemaphoreType.DMA,
            pltpu.SemaphoreType.DMA,
        ],
    )
    def gather(tok_hbm, cb_hbm, out_hbm, idx_v, rows0, rows1, sem0, sem1):
        wid = lax.axis_index("s") * nc + lax.axis_index("c")
        base = wid * b_per_w
        chunk = b_per_w // 4
        pltpu.sync_copy(tok_hbm.at[pl.ds(base, b_per_w)], idx_v)
        rows = (rows0, rows1)
        sems = (sem0, sem1)
        cps = []
        for c in range(4):
            if c >= 2:
                cps[c - 2].wait()
                pltpu.sync_copy(rows[c % 2],
                                out_hbm.at[pl.ds(base + (c - 2) * chunk, chunk)])
            cps.append(pltpu.async_copy(
                cb_hbm.at[idx_v.at[pl.ds(c * chunk, chunk)]],
                rows[c % 2], sems[c % 2]))
        for c in range(2, 4):
            cps[c].wait()
            pltpu.sync_copy(rows[c % 2],
                            out_hbm.at[pl.ds(base + c * chunk, chunk)])

    return gather


def kernel(z, codebook):
    b, d, t = z.shape
    k = codebook.shape[0]
    n = b * t

    bt = 1024
    tok = pl.pallas_call(
        _tok_block_kernel,
        grid=(b, t // bt),
        in_specs=[
            pl.BlockSpec((1, d, bt), lambda i, j: (i, 0, j)),
            pl.BlockSpec((k, d), lambda i, j: (0, 0)),
        ],
        out_specs=pl.BlockSpec((1, 1, bt),
                               lambda i, j, _tb=t // bt: (i * _tb + j, 0, 0)),
        out_shape=jax.ShapeDtypeStruct((n // bt, 1, bt), jnp.int32),
        scratch_shapes=[pltpu.VMEM((k, 1), jnp.float32)],
    )(z, codebook)

    cb_pad = jnp.pad(codebook, ((0, 0), (0, 128 - d)))
    q = _make_sc_gather(n, d, k)(tok.reshape(n), cb_pad)[:, :d]
    return tok.reshape(b, t), q.reshape(b, t, d)


# speedup vs baseline: 3.1552x; 1.0443x over previous
import functools

import jax
import jax.numpy as jnp
from jax import lax
from jax.experimental import pallas as pl
from jax.experimental.pallas import tpu as pltpu
from jax.experimental.pallas import tpu_sc as plsc


def _tok_block_kernel(z_ref, cb_ref, tok_ref, ksq_ref):
    @pl.when((pl.program_id(0) == 0) & (pl.program_id(1) == 0))
    def _():
        cb0 = cb_ref[...]
        ksq_ref[...] = jnp.sum(cb0 * cb0, axis=1, keepdims=True)

    zb = z_ref[0]
    p = jnp.dot(cb_ref[...], zb, preferred_element_type=jnp.float32)
    s = zb * zb
    w = s.shape[0] // 2
    while w >= 1:
        s = (jax.lax.slice_in_dim(s, 0, w, axis=0)
             + jax.lax.slice_in_dim(s, w, 2 * w, axis=0))
        w //= 2
    d = (s - 2.0 * p) + ksq_ref[...]
    m = jnp.min(d, axis=0, keepdims=True)
    kk = d.shape[0]
    iota0 = jax.lax.broadcasted_iota(
        jnp.int32, d.shape, 0).astype(jnp.float32)
    tok_ref[0] = jnp.min(jnp.where(d == m, iota0, float(kk)), axis=0,
                         keepdims=True).astype(jnp.int32)


def _make_sc_gather(n, d, k):
    info = plsc.get_sparse_core_info()
    nc, ns, nl = info.num_cores, info.num_subcores, info.num_lanes
    nw = nc * ns
    assert d % nl == 0 and n % (8 * nw) == 0
    b_per_w = n // nw
    mesh = plsc.VectorSubcoreMesh(core_axis_name="c", subcore_axis_name="s")

    @functools.partial(
        pl.kernel,
        mesh=mesh,
        out_type=jax.ShapeDtypeStruct((n, 128), jnp.float32),
        scratch_types=[
            pltpu.VMEM((b_per_w,), jnp.int32),
            pltpu.VMEM((b_per_w // 4, 128), jnp.float32),
            pltpu.VMEM((b_per_w // 4, 128), jnp.float32),
            pltpu.SemaphoreType.DMA,
            pltpu.SemaphoreType.DMA,
        ],
    )
    def gather(tok_hbm, cb_hbm, out_hbm, idx_v, rows0, rows1, sem0, sem1):
        wid = lax.axis_index("s") * nc + lax.axis_index("c")
        base = wid * b_per_w
        chunk = b_per_w // 4
        pltpu.sync_copy(tok_hbm.at[pl.ds(base, b_per_w)], idx_v)
        rows = (rows0, rows1)
        sems = (sem0, sem1)
        cps = []
        for c in range(4):
            if c >= 2:
                cps[c - 2].wait()
                pltpu.sync_copy(rows[c % 2],
                                out_hbm.at[pl.ds(base + (c - 2) * chunk, chunk)])
            cps.append(pltpu.async_copy(
                cb_hbm.at[idx_v.at[pl.ds(c * chunk, chunk)]],
                rows[c % 2], sems[c % 2]))
        for c in range(2, 4):
            cps[c].wait()
            pltpu.sync_copy(rows[c % 2],
                            out_hbm.at[pl.ds(base + c * chunk, chunk)])

    return gather


def kernel(z, codebook):
    b, d, t = z.shape
    k = codebook.shape[0]

    bt = 2048
    nslab = 2
    ts = t // nslab
    ns = b * ts
    cb_pad = jnp.pad(codebook, ((0, 0), (0, 128 - d)))
    gather = _make_sc_gather(ns, d, k)

    toks = []
    qs = []
    for s in range(nslab):
        tok_s = pl.pallas_call(
            _tok_block_kernel,
            grid=(b, ts // bt),
            in_specs=[
                pl.BlockSpec((1, d, bt),
                             lambda i, j, _s=s, _tb=ts // bt: (i, 0, j + _s * _tb)),
                pl.BlockSpec((k, d), lambda i, j: (0, 0)),
            ],
            out_specs=pl.BlockSpec((1, 1, bt),
                                   lambda i, j, _tb=ts // bt: (i * _tb + j, 0, 0)),
            out_shape=jax.ShapeDtypeStruct((ns // bt, 1, bt), jnp.int32),
            scratch_shapes=[pltpu.VMEM((k, 1), jnp.float32)],
        )(z, codebook)
        toks.append(tok_s.reshape(b, ts))
        qs.append(gather(tok_s.reshape(ns), cb_pad)[:, :d].reshape(b, ts, d))

    tok = jnp.concatenate(toks, axis=1)
    q = jnp.concatenate(qs, axis=1)
    return tok, q
